# baseline (speedup 1.0000x reference)
import functools

import jax
import jax.numpy as jnp
from jax import lax
from jax.experimental import pallas as pl
from jax.experimental.pallas import tpu as pltpu
from jax.experimental.pallas import tpu_sc as plsc

B = 16384
EMB = 16
CONTENT_DIM = 128
H1, H2 = 64, 32
N_ROWS = 1000000

_info = plsc.get_sparse_core_info()
_NC, _NS = _info.num_cores, _info.num_subcores
_NW = _NC * _NS
_BPW = B // _NW

_NSLAB = N_ROWS // 8
_CBLK = 1000

_CH = 64
_NCHUNK = _BPW // _CH


def _tc_compact(tables):
    nblk = _NSLAB // _CBLK

    def body(*refs):
        ins, outs = refs[:32], refs[32:]
        for t in range(4):
            dst = outs[t]
            dst[...] = jnp.concatenate(
                [ins[t * 8 + r][...] for r in range(8)], axis=1)

    in_specs = []
    for _ in range(4):
        for r in range(8):
            in_specs.append(
                pl.BlockSpec((_CBLK, EMB),
                             lambda i, rr=r: (i + rr * nblk, 0)))

    args = []
    for t in tables:
        args.extend([t] * 8)

    return pl.pallas_call(
        body,
        grid=(nblk,),
        in_specs=in_specs,
        out_specs=[pl.BlockSpec((_CBLK, 128), lambda i: (i, 0))] * 4,
        out_shape=[jax.ShapeDtypeStruct((_NSLAB, 128), jnp.float32)] * 4,
    )(*args)


def _sc_gather(users, items, slab_tables):
    mesh = plsc.VectorSubcoreMesh(core_axis_name="c", subcore_axis_name="s")

    @functools.partial(
        pl.kernel,
        mesh=mesh,
        compiler_params=pltpu.CompilerParams(needs_layout_passes=False),
        out_type=[jax.ShapeDtypeStruct((B, EMB), jnp.float32)] * 4,
        scratch_types=[
            pltpu.VMEM((_BPW,), jnp.int32),
            pltpu.VMEM((_BPW,), jnp.int32),
            pltpu.VMEM((_BPW,), jnp.int32),
            pltpu.VMEM((_BPW,), jnp.int32),
            pltpu.VMEM((_CH, 128), jnp.float32),
            pltpu.VMEM((_CH, 128), jnp.float32),
            pltpu.VMEM((_BPW, EMB), jnp.float32),
            pltpu.SemaphoreType.DMA,
            pltpu.SemaphoreType.DMA,
        ],
    )
    def k(uslab_hbm, islab_hbm, urow_hbm, irow_hbm,
          ug_hbm, ig_hbm, um_hbm, im_hbm,
          gu_out, gi_out, mu_out, mi_out,
          uslab_v, islab_v, urow_v, irow_v, buf0, buf1, out_v, sem0, sem1):
        wid = lax.axis_index("s") * _NC + lax.axis_index("c")
        base = wid * _BPW
        pltpu.sync_copy(uslab_hbm.at[pl.ds(base, _BPW)], uslab_v)
        pltpu.sync_copy(islab_hbm.at[pl.ds(base, _BPW)], islab_v)
        pltpu.sync_copy(urow_hbm.at[pl.ds(base, _BPW)], urow_v)
        pltpu.sync_copy(irow_hbm.at[pl.ds(base, _BPW)], irow_v)

        lane = lax.iota(jnp.int32, 16)

        for table_hbm, slab_v, row_v, out_hbm in (
                (ug_hbm, uslab_v, urow_v, gu_out),
                (ig_hbm, islab_v, irow_v, gi_out),
                (um_hbm, uslab_v, urow_v, mu_out),
                (im_hbm, islab_v, irow_v, mi_out)):

            def issue(ch, buf, sem):
                pltpu.async_copy(
                    table_hbm.at[slab_v.at[pl.ds(ch * _CH, _CH)]], buf, sem)

            def drain(ch, buf, sem):
                pltpu.make_async_copy(
                    table_hbm.at[slab_v.at[pl.ds(ch * _CH, _CH)]],
                    buf, sem).wait()

            def extract(ch, buf):
                for g in range(_CH // 16):
                    r16 = row_v[pl.ds(ch * _CH + g * 16, 16)] * EMB
                    srow = g * 16 + lane
                    drow = ch * _CH + g * 16 + lane
                    for c in range(EMB):
                        vals = plsc.load_gather(buf, [srow, r16 + c])
                        plsc.store_scatter(
                            out_v, [drow, jnp.full((16,), c, jnp.int32)],
                            vals)

            issue(0, buf0, sem0)

            def body(g, carry):
                ch0 = g * 2
                issue(ch0 + 1, buf1, sem1)
                drain(ch0, buf0, sem0)
                extract(ch0, buf0)

                @pl.when(ch0 + 2 < _NCHUNK)
                def _():
                    issue(ch0 + 2, buf0, sem0)

                drain(ch0 + 1, buf1, sem1)
                extract(ch0 + 1, buf1)
                return carry

            lax.fori_loop(0, _NCHUNK // 2, body, 0)
            pltpu.sync_copy(out_v, out_hbm.at[pl.ds(base, _BPW)])

    uslab, urow = users % _NSLAB, users // _NSLAB
    islab, irow = items % _NSLAB, items // _NSLAB
    return k(uslab, islab, urow, irow, *slab_tables)


def _tc_mlp(content_vec, gu, gi, mu, mi, W_content, W1, b1, W2, b2, W_out):
    WcT = W_content.T
    W1T = W1.T
    W1u, W1i, W1p = W1T[0:EMB], W1T[EMB:2 * EMB], W1T[2 * EMB:3 * EMB]
    W2T = W2.T
    wg = W_out[:, 0:EMB]
    wh = W_out[:, EMB:]
    b1r = b1.reshape(1, H1)
    b2r = b2.reshape(1, H2)

    BLK = 2048
    grid = (B // BLK,)
    row = lambda i: (i, 0)
    rep = lambda i: (0, 0)

    def body(c_ref, gu_ref, gi_ref, mu_ref, mi_ref,
             wc_ref, w1u_ref, w1i_ref, w1p_ref, b1_ref, w2_ref, b2_ref,
             wg_ref, wh_ref, out_ref):
        proj = jnp.maximum(
            jnp.dot(c_ref[...], wc_ref[...],
                    preferred_element_type=jnp.float32), 0.0)
        pre1 = (jnp.dot(mu_ref[...], w1u_ref[...],
                        preferred_element_type=jnp.float32)
                + jnp.dot(mi_ref[...], w1i_ref[...],
                          preferred_element_type=jnp.float32)
                + jnp.dot(proj, w1p_ref[...],
                          preferred_element_type=jnp.float32)
                + b1_ref[...])
        h1 = jnp.maximum(pre1, 0.0)
        h2 = jnp.maximum(
            jnp.dot(h1, w2_ref[...],
                    preferred_element_type=jnp.float32) + b2_ref[...],
            0.0)
        gmf = gu_ref[...] * gi_ref[...]
        logits = (jnp.sum(gmf * wg_ref[...], axis=1, keepdims=True)
                  + jnp.sum(h2 * wh_ref[...], axis=1, keepdims=True))
        out_ref[...] = jax.nn.sigmoid(logits)

    out = pl.pallas_call(
        body,
        grid=grid,
        in_specs=[
            pl.BlockSpec((BLK, CONTENT_DIM), row),
            pl.BlockSpec((BLK, EMB), row),
            pl.BlockSpec((BLK, EMB), row),
            pl.BlockSpec((BLK, EMB), row),
            pl.BlockSpec((BLK, EMB), row),
            pl.BlockSpec((CONTENT_DIM, EMB), rep),
            pl.BlockSpec((EMB, H1), rep),
            pl.BlockSpec((EMB, H1), rep),
            pl.BlockSpec((EMB, H1), rep),
            pl.BlockSpec((1, H1), rep),
            pl.BlockSpec((H1, H2), rep),
            pl.BlockSpec((1, H2), rep),
            pl.BlockSpec((1, EMB), rep),
            pl.BlockSpec((1, H2), rep),
        ],
        out_specs=pl.BlockSpec((BLK, 1), row),
        out_shape=jax.ShapeDtypeStruct((B, 1), jnp.float32),
    )(content_vec, gu, gi, mu, mi, WcT, W1u, W1i, W1p, b1r, W2T, b2r, wg, wh)
    return out[:, 0]


def kernel(users, items, content_vec, user_gmf, item_gmf, user_mlp, item_mlp,
           W_content, W1, b1, W2, b2, W_out):
    users = users.astype(jnp.int32)
    items = items.astype(jnp.int32)
    slab_tables = _tc_compact((user_gmf, item_gmf, user_mlp, item_mlp))
    gu, gi, mu, mi = _sc_gather(users, items, slab_tables)
    return _tc_mlp(content_vec, gu, gi, mu, mi, W_content, W1, b1, W2, b2, W_out)


# speedup vs baseline: 1.1583x; 1.1583x over previous
import functools

import jax
import jax.numpy as jnp
from jax import lax
from jax.experimental import pallas as pl
from jax.experimental.pallas import tpu as pltpu
from jax.experimental.pallas import tpu_sc as plsc

B = 16384
EMB = 16
CONTENT_DIM = 128
H1, H2 = 64, 32
N_ROWS = 1000000

_info = plsc.get_sparse_core_info()
_NC, _NS = _info.num_cores, _info.num_subcores
_NW = _NC * _NS
_BPW = B // _NW

_NSLAB = N_ROWS // 8
_CBLK = 1000

_CH = 64
_NCHUNK = _BPW // _CH


def _tc_compact(tables):
    nblk = _NSLAB // _CBLK

    def body(*refs):
        ins, outs = refs[:32], refs[32:]
        for t in range(4):
            dst = outs[t]
            dst[...] = jnp.concatenate(
                [ins[t * 8 + r][...] for r in range(8)], axis=1)

    in_specs = []
    for _ in range(4):
        for r in range(8):
            in_specs.append(
                pl.BlockSpec((_CBLK, EMB),
                             lambda i, rr=r: (i + rr * nblk, 0)))

    args = []
    for t in tables:
        args.extend([t] * 8)

    return pl.pallas_call(
        body,
        grid=(nblk,),
        in_specs=in_specs,
        out_specs=[pl.BlockSpec((_CBLK, 128), lambda i: (i, 0))] * 4,
        out_shape=[jax.ShapeDtypeStruct((_NSLAB, 128), jnp.float32)] * 4,
    )(*args)


def _sc_gather(users, items, slab_tables):
    mesh = plsc.VectorSubcoreMesh(core_axis_name="c", subcore_axis_name="s")

    @functools.partial(
        pl.kernel,
        mesh=mesh,
        compiler_params=pltpu.CompilerParams(needs_layout_passes=False),
        out_type=[jax.ShapeDtypeStruct((B, EMB), jnp.float32)] * 4,
        scratch_types=[
            pltpu.VMEM((_BPW,), jnp.int32),
            pltpu.VMEM((_BPW,), jnp.int32),
            pltpu.VMEM((_BPW,), jnp.int32),
            pltpu.VMEM((_BPW,), jnp.int32),
            pltpu.VMEM((_CH, 128), jnp.float32),
            pltpu.VMEM((_CH, 128), jnp.float32),
            pltpu.VMEM((_BPW, EMB), jnp.float32),
            pltpu.SemaphoreType.DMA,
            pltpu.SemaphoreType.DMA,
        ],
    )
    def k(uslab_hbm, islab_hbm, urow_hbm, irow_hbm,
          ug_hbm, ig_hbm, um_hbm, im_hbm,
          gu_out, gi_out, mu_out, mi_out,
          uslab_v, islab_v, urow_v, irow_v, buf0, buf1, out_v, sem0, sem1):
        wid = lax.axis_index("s") * _NC + lax.axis_index("c")
        base = wid * _BPW
        pltpu.sync_copy(uslab_hbm.at[pl.ds(base, _BPW)], uslab_v)
        pltpu.sync_copy(islab_hbm.at[pl.ds(base, _BPW)], islab_v)
        pltpu.sync_copy(urow_hbm.at[pl.ds(base, _BPW)], urow_v)
        pltpu.sync_copy(irow_hbm.at[pl.ds(base, _BPW)], irow_v)

        lane = lax.iota(jnp.int32, 16)

        for table_hbm, slab_v, row_v, out_hbm in (
                (ug_hbm, uslab_v, urow_v, gu_out),
                (ig_hbm, islab_v, irow_v, gi_out),
                (um_hbm, uslab_v, urow_v, mu_out),
                (im_hbm, islab_v, irow_v, mi_out)):

            def issue(ch, buf, sem):
                pltpu.async_copy(
                    table_hbm.at[slab_v.at[pl.ds(ch * _CH, _CH)]], buf, sem)

            def drain(ch, buf, sem):
                pltpu.make_async_copy(
                    table_hbm.at[slab_v.at[pl.ds(ch * _CH, _CH)]],
                    buf, sem).wait()

            def extract(ch, buf):
                for g in range(_CH // 16):
                    r16 = row_v[pl.ds(ch * _CH + g * 16, 16)] * EMB
                    srow = g * 16 + lane
                    drow = ch * _CH + g * 16 + lane
                    for c in range(EMB):
                        vals = plsc.load_gather(buf, [srow, r16 + c])
                        plsc.store_scatter(
                            out_v, [drow, jnp.full((16,), c, jnp.int32)],
                            vals)

            issue(0, buf0, sem0)

            def body(g, carry):
                ch0 = g * 2
                issue(ch0 + 1, buf1, sem1)
                drain(ch0, buf0, sem0)
                extract(ch0, buf0)

                @pl.when(ch0 + 2 < _NCHUNK)
                def _():
                    issue(ch0 + 2, buf0, sem0)

                drain(ch0 + 1, buf1, sem1)
                extract(ch0 + 1, buf1)
                return carry

            lax.fori_loop(0, _NCHUNK // 2, body, 0)
            pltpu.sync_copy(out_v, out_hbm.at[pl.ds(base, _BPW)])

    uslab, urow = users >> 3, users & 7
    islab, irow = items >> 3, items & 7
    return k(uslab, islab, urow, irow, *slab_tables)


def _tc_mlp(content_vec, gu, gi, mu, mi, W_content, W1, b1, W2, b2, W_out):
    WcT = W_content.T
    W1T = W1.T
    W1u, W1i, W1p = W1T[0:EMB], W1T[EMB:2 * EMB], W1T[2 * EMB:3 * EMB]
    W2T = W2.T
    wg = W_out[:, 0:EMB]
    wh = W_out[:, EMB:]
    b1r = b1.reshape(1, H1)
    b2r = b2.reshape(1, H2)

    BLK = 2048
    grid = (B // BLK,)
    row = lambda i: (i, 0)
    rep = lambda i: (0, 0)

    def body(c_ref, gu_ref, gi_ref, mu_ref, mi_ref,
             wc_ref, w1u_ref, w1i_ref, w1p_ref, b1_ref, w2_ref, b2_ref,
             wg_ref, wh_ref, out_ref):
        proj = jnp.maximum(
            jnp.dot(c_ref[...], wc_ref[...],
                    preferred_element_type=jnp.float32), 0.0)
        pre1 = (jnp.dot(mu_ref[...], w1u_ref[...],
                        preferred_element_type=jnp.float32)
                + jnp.dot(mi_ref[...], w1i_ref[...],
                          preferred_element_type=jnp.float32)
                + jnp.dot(proj, w1p_ref[...],
                          preferred_element_type=jnp.float32)
                + b1_ref[...])
        h1 = jnp.maximum(pre1, 0.0)
        h2 = jnp.maximum(
            jnp.dot(h1, w2_ref[...],
                    preferred_element_type=jnp.float32) + b2_ref[...],
            0.0)
        gmf = gu_ref[...] * gi_ref[...]
        logits = (jnp.sum(gmf * wg_ref[...], axis=1, keepdims=True)
                  + jnp.sum(h2 * wh_ref[...], axis=1, keepdims=True))
        out_ref[...] = jax.nn.sigmoid(logits)

    out = pl.pallas_call(
        body,
        grid=grid,
        in_specs=[
            pl.BlockSpec((BLK, CONTENT_DIM), row),
            pl.BlockSpec((BLK, EMB), row),
            pl.BlockSpec((BLK, EMB), row),
            pl.BlockSpec((BLK, EMB), row),
            pl.BlockSpec((BLK, EMB), row),
            pl.BlockSpec((CONTENT_DIM, EMB), rep),
            pl.BlockSpec((EMB, H1), rep),
            pl.BlockSpec((EMB, H1), rep),
            pl.BlockSpec((EMB, H1), rep),
            pl.BlockSpec((1, H1), rep),
            pl.BlockSpec((H1, H2), rep),
            pl.BlockSpec((1, H2), rep),
            pl.BlockSpec((1, EMB), rep),
            pl.BlockSpec((1, H2), rep),
        ],
        out_specs=pl.BlockSpec((BLK, 1), row),
        out_shape=jax.ShapeDtypeStruct((B, 1), jnp.float32),
    )(content_vec, gu, gi, mu, mi, WcT, W1u, W1i, W1p, b1r, W2T, b2r, wg, wh)
    return out[:, 0]


def kernel(users, items, content_vec, user_gmf, item_gmf, user_mlp, item_mlp,
           W_content, W1, b1, W2, b2, W_out):
    users = users.astype(jnp.int32)
    items = items.astype(jnp.int32)
    slab_tables = [t.reshape(_NSLAB, 128)
                   for t in (user_gmf, item_gmf, user_mlp, item_mlp)]
    gu, gi, mu, mi = _sc_gather(users, items, slab_tables)
    return _tc_mlp(content_vec, gu, gi, mu, mi, W_content, W1, b1, W2, b2, W_out)


# speedup vs baseline: 1.6630x; 1.4357x over previous
import functools

import jax
import jax.numpy as jnp
from jax import lax
from jax.experimental import pallas as pl
from jax.experimental.pallas import tpu as pltpu
from jax.experimental.pallas import tpu_sc as plsc

B = 16384
EMB = 16
CONTENT_DIM = 128
H1, H2 = 64, 32
N_ROWS = 1000000

_info = plsc.get_sparse_core_info()
_NC, _NS = _info.num_cores, _info.num_subcores
_NW = _NC * _NS
_BPW = B // _NW

_NSLAB = N_ROWS // 8
_CBLK = 1000

_RCH = 128


def _tc_compact(tables):
    nblk = _NSLAB // _CBLK

    def body(*refs):
        ins, outs = refs[:32], refs[32:]
        for t in range(4):
            dst = outs[t]
            dst[...] = jnp.concatenate(
                [ins[t * 8 + r][...] for r in range(8)], axis=1)

    in_specs = []
    for _ in range(4):
        for r in range(8):
            in_specs.append(
                pl.BlockSpec((_CBLK, EMB),
                             lambda i, rr=r: (i + rr * nblk, 0)))

    args = []
    for t in tables:
        args.extend([t] * 8)

    return pl.pallas_call(
        body,
        grid=(nblk,),
        in_specs=in_specs,
        out_specs=[pl.BlockSpec((_CBLK, 128), lambda i: (i, 0))] * 4,
        out_shape=[jax.ShapeDtypeStruct((_NSLAB, 128), jnp.float32)] * 4,
    )(*args)


def _sc_gather(users, items, slab_tables):
    mesh = plsc.VectorSubcoreMesh(core_axis_name="c", subcore_axis_name="s")

    @functools.partial(
        pl.kernel,
        mesh=mesh,
        out_type=[jax.ShapeDtypeStruct((B, EMB), jnp.float32)] * 4,
        scratch_types=[
            pltpu.VMEM((_BPW,), jnp.int32),
            pltpu.VMEM((_BPW,), jnp.int32),
            pltpu.VMEM((_RCH, EMB), jnp.float32),
            pltpu.VMEM((_RCH, EMB), jnp.float32),
            pltpu.VMEM((_RCH, EMB), jnp.float32),
            pltpu.VMEM((_RCH, EMB), jnp.float32),
            pltpu.SemaphoreType.DMA,
        ],
    )
    def k(users_hbm, items_hbm,
          ug_hbm, ig_hbm, um_hbm, im_hbm,
          gu_out, gi_out, mu_out, mi_out,
          uidx, iidx, bgu, bgi, bmu, bmi, sem):
        wid = lax.axis_index("s") * _NC + lax.axis_index("c")
        base = wid * _BPW
        pltpu.sync_copy(users_hbm.at[pl.ds(base, _BPW)], uidx)
        pltpu.sync_copy(items_hbm.at[pl.ds(base, _BPW)], iidx)

        def chunk(ch, carry):
            def issue(g, carry2):
                uvec = uidx[pl.ds(ch * _RCH + g * 16, 16)]
                ivec = iidx[pl.ds(ch * _RCH + g * 16, 16)]
                for j in range(16):
                    u = uvec[j]
                    v = ivec[j]
                    d = g * 16 + j
                    pltpu.async_copy(ug_hbm.at[u], bgu.at[d], sem)
                    pltpu.async_copy(um_hbm.at[u], bmu.at[d], sem)
                    pltpu.async_copy(ig_hbm.at[v], bgi.at[d], sem)
                    pltpu.async_copy(im_hbm.at[v], bmi.at[d], sem)
                return carry2

            lax.fori_loop(0, _RCH // 16, issue, 0)
            for buf, table in ((bgu, ug_hbm), (bgi, ig_hbm),
                               (bmu, um_hbm), (bmi, im_hbm)):
                pltpu.make_async_copy(table.at[pl.ds(0, _RCH)],
                                      buf, sem).wait()
            dst = base + ch * _RCH
            pltpu.sync_copy(bgu, gu_out.at[pl.ds(dst, _RCH)])
            pltpu.sync_copy(bgi, gi_out.at[pl.ds(dst, _RCH)])
            pltpu.sync_copy(bmu, mu_out.at[pl.ds(dst, _RCH)])
            pltpu.sync_copy(bmi, mi_out.at[pl.ds(dst, _RCH)])
            return carry

        lax.fori_loop(0, _BPW // _RCH, chunk, 0)

    return k(users, items, *slab_tables)


def _tc_mlp(content_vec, gu, gi, mu, mi, W_content, W1, b1, W2, b2, W_out):
    WcT = W_content.T
    W1T = W1.T
    W1u, W1i, W1p = W1T[0:EMB], W1T[EMB:2 * EMB], W1T[2 * EMB:3 * EMB]
    W2T = W2.T
    wg = W_out[:, 0:EMB]
    wh = W_out[:, EMB:]
    b1r = b1.reshape(1, H1)
    b2r = b2.reshape(1, H2)

    BLK = 2048
    grid = (B // BLK,)
    row = lambda i: (i, 0)
    rep = lambda i: (0, 0)

    def body(c_ref, gu_ref, gi_ref, mu_ref, mi_ref,
             wc_ref, w1u_ref, w1i_ref, w1p_ref, b1_ref, w2_ref, b2_ref,
             wg_ref, wh_ref, out_ref):
        proj = jnp.maximum(
            jnp.dot(c_ref[...], wc_ref[...],
                    preferred_element_type=jnp.float32), 0.0)
        pre1 = (jnp.dot(mu_ref[...], w1u_ref[...],
                        preferred_element_type=jnp.float32)
                + jnp.dot(mi_ref[...], w1i_ref[...],
                          preferred_element_type=jnp.float32)
                + jnp.dot(proj, w1p_ref[...],
                          preferred_element_type=jnp.float32)
                + b1_ref[...])
        h1 = jnp.maximum(pre1, 0.0)
        h2 = jnp.maximum(
            jnp.dot(h1, w2_ref[...],
                    preferred_element_type=jnp.float32) + b2_ref[...],
            0.0)
        gmf = gu_ref[...] * gi_ref[...]
        logits = (jnp.sum(gmf * wg_ref[...], axis=1, keepdims=True)
                  + jnp.sum(h2 * wh_ref[...], axis=1, keepdims=True))
        out_ref[...] = jax.nn.sigmoid(logits)

    out = pl.pallas_call(
        body,
        grid=grid,
        in_specs=[
            pl.BlockSpec((BLK, CONTENT_DIM), row),
            pl.BlockSpec((BLK, EMB), row),
            pl.BlockSpec((BLK, EMB), row),
            pl.BlockSpec((BLK, EMB), row),
            pl.BlockSpec((BLK, EMB), row),
            pl.BlockSpec((CONTENT_DIM, EMB), rep),
            pl.BlockSpec((EMB, H1), rep),
            pl.BlockSpec((EMB, H1), rep),
            pl.BlockSpec((EMB, H1), rep),
            pl.BlockSpec((1, H1), rep),
            pl.BlockSpec((H1, H2), rep),
            pl.BlockSpec((1, H2), rep),
            pl.BlockSpec((1, EMB), rep),
            pl.BlockSpec((1, H2), rep),
        ],
        out_specs=pl.BlockSpec((BLK, 1), row),
        out_shape=jax.ShapeDtypeStruct((B, 1), jnp.float32),
    )(content_vec, gu, gi, mu, mi, WcT, W1u, W1i, W1p, b1r, W2T, b2r, wg, wh)
    return out[:, 0]


def kernel(users, items, content_vec, user_gmf, item_gmf, user_mlp, item_mlp,
           W_content, W1, b1, W2, b2, W_out):
    users = users.astype(jnp.int32)
    items = items.astype(jnp.int32)
    gu, gi, mu, mi = _sc_gather(
        users, items, (user_gmf, item_gmf, user_mlp, item_mlp))
    return _tc_mlp(content_vec, gu, gi, mu, mi, W_content, W1, b1, W2, b2, W_out)
